# baseline (speedup 1.0000x reference)
import functools
import jax
import jax.numpy as jnp
from jax import lax
from jax.experimental import pallas as pl
from jax.experimental.pallas import tpu as pltpu
from jax.experimental.pallas import tpu_sc as plsc

THR = 0.9
_CH = 10000


def kernel(nidx, score, specweights, tidxs):
    V, K = nidx.shape
    nidx_flat = jnp.ravel(nidx.T)
    score_flat = jnp.ravel(jnp.transpose(score, (1, 2, 0)))
    info = plsc.get_sparse_core_info()
    nc, ns = info.num_cores, info.num_subcores
    nw = nc * ns
    rows_per_w = K // nw
    nch = V // _CH

    mesh = plsc.VectorSubcoreMesh(core_axis_name="c", subcore_axis_name="s")

    @functools.partial(
        pl.kernel,
        mesh=mesh,
        out_type=jax.ShapeDtypeStruct((K * V,), jnp.int32),
        scratch_types=[
            pltpu.VMEM((_CH,), jnp.int32),
            pltpu.VMEM((_CH,), jnp.float32),
            pltpu.VMEM((_CH,), jnp.int32),
        ],
    )
    def sc_run(nidx_hbm, score_hbm, out_hbm, nbuf, sbuf, obuf):
        wid = lax.axis_index("s") * nc + lax.axis_index("c")
        for r in range(rows_per_w):
            k = wid * rows_per_w + r

            def chunk_body(c, _):
                off = c * _CH

                @pl.when(k == 0)
                def _copy_row():
                    pltpu.sync_copy(nidx_hbm.at[pl.ds(k * V + off, _CH)], obuf)

                @pl.when(k > 0)
                def _mask_row():
                    pltpu.sync_copy(nidx_hbm.at[pl.ds(k * V + off, _CH)], nbuf)
                    pltpu.sync_copy(
                        score_hbm.at[pl.ds((k - 1) * V + off, _CH)], sbuf)

                    def vec_body(i, _):
                        s = sbuf[pl.ds(i * 16, 16)]
                        n = nbuf[pl.ds(i * 16, 16)]
                        obuf[pl.ds(i * 16, 16)] = jnp.where(
                            s >= THR, n, jnp.full((16,), -1, jnp.int32))
                        return 0

                    lax.fori_loop(0, _CH // 16, vec_body, 0)

                pltpu.sync_copy(obuf, out_hbm.at[pl.ds(k * V + off, _CH)])
                return 0

            lax.fori_loop(0, nch, chunk_body, 0)

    out_flat = sc_run(nidx_flat, score_flat)
    return out_flat.reshape(K, V).T


# speedup vs baseline: 16.5776x; 16.5776x over previous
import jax
import jax.numpy as jnp
from jax.experimental import pallas as pl
from jax.experimental.pallas import tpu as pltpu

THR = 0.9
_BL = 16768


def _body(nidx_ref, score_ref, out_ref):
    n = nidx_ref[...]
    s3 = score_ref[...]
    s = s3.reshape(s3.shape[0], s3.shape[2])
    ones = jnp.ones((1, s.shape[1]), dtype=jnp.float32)
    full = jnp.concatenate([ones, s], axis=0)
    out_ref[...] = jnp.where(full < THR, -1, n)


def kernel(nidx, score, specweights, tidxs):
    V, K = nidx.shape
    nidx_t = nidx.T
    score_t = jnp.transpose(score, (1, 2, 0))
    nb = pl.cdiv(V, _BL)
    out_t = pl.pallas_call(
        _body,
        grid=(nb,),
        in_specs=[
            pl.BlockSpec((K, _BL), lambda i: (0, i)),
            pl.BlockSpec((K - 1, 1, _BL), lambda i: (0, 0, i)),
        ],
        out_specs=pl.BlockSpec((K, _BL), lambda i: (0, i)),
        out_shape=jax.ShapeDtypeStruct((K, V), jnp.int32),
        compiler_params=pltpu.CompilerParams(
            dimension_semantics=("parallel",),
        ),
    )(nidx_t, score_t)
    return out_t.T
